# baseline (speedup 1.0000x reference)
import jax
import jax.numpy as jnp
from jax import lax
from jax.experimental import pallas as pl
from jax.experimental.pallas import tpu as pltpu
from jax.experimental.pallas import tpu_sc as plsc

N = 10000
DF = 128
H = 16
C = 7
E = 320000

NC = 2
NS = 16
CHUNK = 2000
CPW = 5
EPW = CHUNK * CPW
N_PAD = 10240
RPS = N_PAD // NS

ROWS = 1024
ROWS8 = ROWS // 8
GRID_TC = N_PAD // ROWS
NP8 = N_PAD * H // 128

_mesh = plsc.VectorSubcoreMesh(core_axis_name="c", subcore_axis_name="s")
_sc_params = pltpu.CompilerParams(use_tc_tiling_on_sc=False)


def _deg_body(dst_hbm, ones_hbm, out_hbm, dst_v, ones_v, acc_sh, sem):
    cid = lax.axis_index("c")
    sid = lax.axis_index("s")
    w = sid * NC + cid
    pltpu.sync_copy(ones_hbm.at[pl.ds(sid * RPS, RPS)],
                    acc_sh.at[pl.ds(sid * RPS, RPS)])
    pltpu.sync_copy(ones_hbm.at[pl.ds(0, CHUNK)], ones_v)
    pltpu.sync_copy(dst_hbm.at[pl.ds(w * EPW, EPW)], dst_v)
    plsc.subcore_barrier()

    @pl.loop(0, CPW)
    def _(j):
        pltpu.async_copy(ones_v, acc_sh.at[dst_v.at[pl.ds(j * CHUNK, CHUNK)]],
                         sem, add=True)

    @pl.loop(0, CPW)
    def _(j):
        pltpu.make_async_copy(
            ones_v, acc_sh.at[dst_v.at[pl.ds(j * CHUNK, CHUNK)]], sem).wait()

    plsc.subcore_barrier()
    pltpu.sync_copy(acc_sh.at[pl.ds(sid * RPS, RPS)],
                    out_hbm.at[cid, pl.ds(sid * RPS, RPS)])


def _agg_body(g_hbm, src_hbm, dst_hbm, out_hbm,
              src_v, dst_v, r0, r1, g_sh, acc_sh, sg0, sg1):
    cid = lax.axis_index("c")
    sid = lax.axis_index("s")
    w = sid * NC + cid
    pltpu.async_copy(g_hbm.at[pl.ds(sid * RPS, RPS)],
                     g_sh.at[pl.ds(sid * RPS, RPS)], sg1)
    pltpu.async_copy(g_hbm.at[pl.ds(sid * RPS, RPS)],
                     acc_sh.at[pl.ds(sid * RPS, RPS)], sg0)
    pltpu.sync_copy(src_hbm.at[pl.ds(w * EPW, EPW)], src_v)
    pltpu.sync_copy(dst_hbm.at[pl.ds(w * EPW, EPW)], dst_v)
    pltpu.make_async_copy(g_hbm.at[pl.ds(sid * RPS, RPS)],
                          g_sh.at[pl.ds(sid * RPS, RPS)], sg1).wait()
    pltpu.make_async_copy(g_hbm.at[pl.ds(sid * RPS, RPS)],
                          acc_sh.at[pl.ds(sid * RPS, RPS)], sg0).wait()
    plsc.subcore_barrier()

    pltpu.async_copy(g_sh.at[src_v.at[pl.ds(0, CHUNK)]], r0, sg0)

    @pl.loop(0, CPW - 1, step=2)
    def _(j):
        pltpu.async_copy(g_sh.at[src_v.at[pl.ds((j + 1) * CHUNK, CHUNK)]],
                         r1, sg1)
        pltpu.make_async_copy(g_sh.at[src_v.at[pl.ds(j * CHUNK, CHUNK)]],
                              r0, sg0).wait()
        pltpu.sync_copy(r0, acc_sh.at[dst_v.at[pl.ds(j * CHUNK, CHUNK)]],
                        add=True)
        pltpu.async_copy(
            g_sh.at[src_v.at[pl.ds((j + 2) * CHUNK, CHUNK)]], r0, sg0)
        pltpu.make_async_copy(
            g_sh.at[src_v.at[pl.ds((j + 1) * CHUNK, CHUNK)]], r1, sg1).wait()
        pltpu.sync_copy(r1, acc_sh.at[dst_v.at[pl.ds((j + 1) * CHUNK, CHUNK)]],
                        add=True)

    pltpu.make_async_copy(
        g_sh.at[src_v.at[pl.ds((CPW - 1) * CHUNK, CHUNK)]], r0, sg0).wait()
    pltpu.sync_copy(r0, acc_sh.at[dst_v.at[pl.ds((CPW - 1) * CHUNK, CHUNK)]],
                    add=True)

    plsc.subcore_barrier()
    pltpu.sync_copy(acc_sh.at[pl.ds(sid * RPS, RPS)],
                    out_hbm.at[cid, pl.ds(sid * RPS, RPS)])


def _sc_deg(dst, ones2):
    return pl.kernel(
        _deg_body,
        out_type=jax.ShapeDtypeStruct((NC, N_PAD, H), jnp.float32),
        mesh=_mesh,
        scratch_types=[
            pltpu.VMEM((EPW,), jnp.int32),
            pltpu.VMEM((CHUNK, H), jnp.float32),
            pltpu.VMEM_SHARED((N_PAD, H), jnp.float32),
            pltpu.SemaphoreType.DMA,
        ],
        compiler_params=_sc_params,
    )(dst, ones2)


def _sc_agg(g, src, dst):
    return pl.kernel(
        _agg_body,
        out_type=jax.ShapeDtypeStruct((NC, N_PAD, H), jnp.float32),
        mesh=_mesh,
        scratch_types=[
            pltpu.VMEM((EPW,), jnp.int32),
            pltpu.VMEM((EPW,), jnp.int32),
            pltpu.VMEM((CHUNK, H), jnp.float32),
            pltpu.VMEM((CHUNK, H), jnp.float32),
            pltpu.VMEM_SHARED((N_PAD, H), jnp.float32),
            pltpu.VMEM_SHARED((N_PAD, H), jnp.float32),
            pltpu.SemaphoreType.DMA,
            pltpu.SemaphoreType.DMA,
        ],
        compiler_params=_sc_params,
    )(g, src, dst)


def _mm_body(xs_ref, w_ref, h_ref):
    h_ref[...] = jnp.dot(xs_ref[...], w_ref[...],
                         preferred_element_type=jnp.float32)


def _tc_mm(xs, W1s):
    return pl.pallas_call(
        _mm_body,
        grid=(GRID_TC,),
        in_specs=[pl.BlockSpec((ROWS8, 8 * DF), lambda i: (i, 0)),
                  pl.BlockSpec((8 * DF, 128), lambda i: (0, 0))],
        out_specs=pl.BlockSpec((ROWS8, 128), lambda i: (i, 0)),
        out_shape=jax.ShapeDtypeStruct((NP8, 128), jnp.float32),
    )(xs, W1s)


def _s1_body(deg_ref, h_ref, g_ref, dis_ref):
    deg = deg_ref[0] + deg_ref[1] - 1.0
    dis = lax.rsqrt(deg)
    dis_ref[...] = dis
    g_ref[...] = dis * h_ref[...]


def _tc_scale(deg2v, hv):
    return pl.pallas_call(
        _s1_body,
        grid=(GRID_TC,),
        in_specs=[pl.BlockSpec((NC, ROWS8, 128), lambda i: (0, i, 0)),
                  pl.BlockSpec((ROWS8, 128), lambda i: (i, 0))],
        out_specs=[pl.BlockSpec((ROWS8, 128), lambda i: (i, 0)),
                   pl.BlockSpec((ROWS8, 128), lambda i: (i, 0))],
        out_shape=[jax.ShapeDtypeStruct((NP8, 128), jnp.float32),
                   jax.ShapeDtypeStruct((NP8, 128), jnp.float32)],
    )(deg2v, hv)


def _s2_body(a_ref, g1_ref, dis_ref, w2_ref, b1_ref, g2_ref):
    dis = dis_ref[...]
    pre = dis * (a_ref[0] + a_ref[1] - g1_ref[...]) + b1_ref[...]
    h = jnp.maximum(pre, 0.0)
    h2 = jnp.dot(h, w2_ref[...], preferred_element_type=jnp.float32)
    g2_ref[...] = dis * h2


def _tc_layer2(agg1v, g1v, dis, W2bd, b1t):
    return pl.pallas_call(
        _s2_body,
        grid=(GRID_TC,),
        in_specs=[pl.BlockSpec((NC, ROWS8, 128), lambda i: (0, i, 0)),
                  pl.BlockSpec((ROWS8, 128), lambda i: (i, 0)),
                  pl.BlockSpec((ROWS8, 128), lambda i: (i, 0)),
                  pl.BlockSpec((128, 128), lambda i: (0, 0)),
                  pl.BlockSpec((1, 128), lambda i: (0, 0))],
        out_specs=pl.BlockSpec((ROWS8, 128), lambda i: (i, 0)),
        out_shape=jax.ShapeDtypeStruct((NP8, 128), jnp.float32),
    )(agg1v, g1v, dis, W2bd, b1t)


def _s3_body(a_ref, g2_ref, dis_ref, b2_ref, seg_ref, o_ref):
    logits = (dis_ref[...] * (a_ref[0] + a_ref[1] - g2_ref[...])
              + b2_ref[...])
    col = lax.broadcasted_iota(jnp.int32, logits.shape, 1)
    masked = jnp.where(col % H < C, logits, -jnp.inf)
    m = jnp.max(masked, axis=1, keepdims=True)
    e = jnp.exp(masked - m)
    s = jnp.dot(e, seg_ref[...], preferred_element_type=jnp.float32)
    o_ref[...] = logits - (m + jnp.log(s))


def _tc_logsoftmax(agg2v, g2v, dis, b2t, seg):
    return pl.pallas_call(
        _s3_body,
        grid=(GRID_TC,),
        in_specs=[pl.BlockSpec((NC, ROWS8, 128), lambda i: (0, i, 0)),
                  pl.BlockSpec((ROWS8, 128), lambda i: (i, 0)),
                  pl.BlockSpec((ROWS8, 128), lambda i: (i, 0)),
                  pl.BlockSpec((1, 128), lambda i: (0, 0)),
                  pl.BlockSpec((128, 128), lambda i: (0, 0))],
        out_specs=pl.BlockSpec((ROWS8, 128), lambda i: (i, 0)),
        out_shape=jax.ShapeDtypeStruct((NP8, 128), jnp.float32),
    )(agg2v, g2v, dis, b2t, seg)


def kernel(x, edge_index, W1, b1, W2, b2):
    ei32 = edge_index.astype(jnp.int32)
    src = ei32[0]
    dst = ei32[1]

    x16 = jnp.pad(x.astype(jnp.bfloat16), ((0, N_PAD - N), (0, 0)))
    xs = x16.reshape(NP8, 8 * DF)
    eye8 = jnp.eye(8, dtype=jnp.float32)
    W1s = jnp.kron(eye8, W1).astype(jnp.bfloat16)
    W2p = jnp.zeros((H, H), jnp.float32).at[:, :C].set(W2)
    W2bd = jnp.kron(eye8, W2p)
    ones2 = jnp.ones((N_PAD, H), jnp.float32)
    b1t = jnp.tile(b1, 8).reshape(1, 128)
    b2t = jnp.tile(jnp.zeros((H,), jnp.float32).at[:C].set(b2), 8)
    b2t = b2t.reshape(1, 128)
    lane = jnp.arange(128)
    seg = (lane[:, None] // H == lane[None, :] // H).astype(jnp.float32)

    deg2 = _sc_deg(dst, ones2)
    hv = _tc_mm(xs, W1s)
    deg2v = deg2.reshape(NC, NP8, 128)
    g1v, dis = _tc_scale(deg2v, hv)
    agg1 = _sc_agg(g1v.reshape(N_PAD, H), src, dst)
    g2v = _tc_layer2(agg1.reshape(NC, NP8, 128), g1v, dis, W2bd, b1t)
    agg2 = _sc_agg(g2v.reshape(N_PAD, H), src, dst)
    out128 = _tc_logsoftmax(agg2.reshape(NC, NP8, 128), g2v, dis, b2t, seg)
    return out128.reshape(N_PAD, H)[:N, :C]


# speedup vs baseline: 1.1006x; 1.1006x over previous
import jax
import jax.numpy as jnp
from jax import lax
from jax.experimental import pallas as pl
from jax.experimental.pallas import tpu as pltpu
from jax.experimental.pallas import tpu_sc as plsc

N = 10000
DF = 128
H = 16
C = 7
E = 320000

NC = 2
NS = 16
CHUNK = 2000
CPW = 5
EPW = CHUNK * CPW
N_PAD = 10240
RPS = N_PAD // NS

ROWS = 1024
ROWS8 = ROWS // 8
GRID_TC = N_PAD // ROWS
NP8 = N_PAD * H // 128

_mesh = plsc.VectorSubcoreMesh(core_axis_name="c", subcore_axis_name="s")
_sc_params = pltpu.CompilerParams(use_tc_tiling_on_sc=False)


def _deg_body(ei_hbm, ones_hbm, out_hbm, dst_v, ones_v, acc_sh, sem):
    cid = lax.axis_index("c")
    sid = lax.axis_index("s")
    w = sid * NC + cid
    pltpu.sync_copy(ones_hbm.at[pl.ds(sid * RPS, RPS)],
                    acc_sh.at[pl.ds(sid * RPS, RPS)])
    pltpu.sync_copy(ones_hbm.at[pl.ds(0, CHUNK)], ones_v)
    pltpu.sync_copy(ei_hbm.at[1, pl.ds(w * EPW, EPW)], dst_v)
    plsc.subcore_barrier()

    @pl.loop(0, CPW)
    def _(j):
        pltpu.async_copy(ones_v, acc_sh.at[dst_v.at[pl.ds(j * CHUNK, CHUNK)]],
                         sem, add=True)

    @pl.loop(0, CPW)
    def _(j):
        pltpu.make_async_copy(
            ones_v, acc_sh.at[dst_v.at[pl.ds(j * CHUNK, CHUNK)]], sem).wait()

    plsc.subcore_barrier()
    pltpu.sync_copy(acc_sh.at[pl.ds(sid * RPS, RPS)],
                    out_hbm.at[cid, pl.ds(sid * RPS, RPS)])


def _agg_body(g_hbm, ei_hbm, out_hbm,
              src_v, dst_v, r0, r1, g_sh, acc_sh, sg0, sg1):
    cid = lax.axis_index("c")
    sid = lax.axis_index("s")
    w = sid * NC + cid
    pltpu.async_copy(g_hbm.at[pl.ds(sid * RPS, RPS)],
                     g_sh.at[pl.ds(sid * RPS, RPS)], sg1)
    pltpu.async_copy(g_hbm.at[pl.ds(sid * RPS, RPS)],
                     acc_sh.at[pl.ds(sid * RPS, RPS)], sg0)
    pltpu.sync_copy(ei_hbm.at[0, pl.ds(w * EPW, EPW)], src_v)
    pltpu.sync_copy(ei_hbm.at[1, pl.ds(w * EPW, EPW)], dst_v)
    pltpu.make_async_copy(g_hbm.at[pl.ds(sid * RPS, RPS)],
                          g_sh.at[pl.ds(sid * RPS, RPS)], sg1).wait()
    pltpu.make_async_copy(g_hbm.at[pl.ds(sid * RPS, RPS)],
                          acc_sh.at[pl.ds(sid * RPS, RPS)], sg0).wait()
    plsc.subcore_barrier()

    pltpu.async_copy(g_sh.at[src_v.at[pl.ds(0, CHUNK)]], r0, sg0)

    @pl.loop(0, CPW - 1, step=2)
    def _(j):
        pltpu.async_copy(g_sh.at[src_v.at[pl.ds((j + 1) * CHUNK, CHUNK)]],
                         r1, sg1)
        pltpu.make_async_copy(g_sh.at[src_v.at[pl.ds(j * CHUNK, CHUNK)]],
                              r0, sg0).wait()
        pltpu.sync_copy(r0, acc_sh.at[dst_v.at[pl.ds(j * CHUNK, CHUNK)]],
                        add=True)
        pltpu.async_copy(
            g_sh.at[src_v.at[pl.ds((j + 2) * CHUNK, CHUNK)]], r0, sg0)
        pltpu.make_async_copy(
            g_sh.at[src_v.at[pl.ds((j + 1) * CHUNK, CHUNK)]], r1, sg1).wait()
        pltpu.sync_copy(r1, acc_sh.at[dst_v.at[pl.ds((j + 1) * CHUNK, CHUNK)]],
                        add=True)

    pltpu.make_async_copy(
        g_sh.at[src_v.at[pl.ds((CPW - 1) * CHUNK, CHUNK)]], r0, sg0).wait()
    pltpu.sync_copy(r0, acc_sh.at[dst_v.at[pl.ds((CPW - 1) * CHUNK, CHUNK)]],
                    add=True)

    plsc.subcore_barrier()
    pltpu.sync_copy(acc_sh.at[pl.ds(sid * RPS, RPS)],
                    out_hbm.at[cid, pl.ds(sid * RPS, RPS)])


def _sc_deg(ei32, ones2):
    return pl.kernel(
        _deg_body,
        out_type=jax.ShapeDtypeStruct((NC, N_PAD, H), jnp.float32),
        mesh=_mesh,
        scratch_types=[
            pltpu.VMEM((EPW,), jnp.int32),
            pltpu.VMEM((CHUNK, H), jnp.float32),
            pltpu.VMEM_SHARED((N_PAD, H), jnp.float32),
            pltpu.SemaphoreType.DMA,
        ],
        compiler_params=_sc_params,
    )(ei32, ones2)


def _sc_agg(g, ei32):
    return pl.kernel(
        _agg_body,
        out_type=jax.ShapeDtypeStruct((NC, N_PAD, H), jnp.float32),
        mesh=_mesh,
        scratch_types=[
            pltpu.VMEM((EPW,), jnp.int32),
            pltpu.VMEM((EPW,), jnp.int32),
            pltpu.VMEM((CHUNK, H), jnp.float32),
            pltpu.VMEM((CHUNK, H), jnp.float32),
            pltpu.VMEM_SHARED((N_PAD, H), jnp.float32),
            pltpu.VMEM_SHARED((N_PAD, H), jnp.float32),
            pltpu.SemaphoreType.DMA,
            pltpu.SemaphoreType.DMA,
        ],
        compiler_params=_sc_params,
    )(g, ei32)


def _mm_body(xs_ref, w_ref, h_ref):
    h_ref[...] = jnp.dot(xs_ref[...], w_ref[...],
                         preferred_element_type=jnp.float32)


def _tc_mm(xs, W1s):
    return pl.pallas_call(
        _mm_body,
        grid=(GRID_TC,),
        in_specs=[pl.BlockSpec((ROWS8, 8 * DF), lambda i: (i, 0)),
                  pl.BlockSpec((8 * DF, 128), lambda i: (0, 0))],
        out_specs=pl.BlockSpec((ROWS8, 128), lambda i: (i, 0)),
        out_shape=jax.ShapeDtypeStruct((NP8, 128), jnp.float32),
    )(xs, W1s)


def _s1_body(deg_ref, h_ref, g_ref, dis_ref):
    deg = deg_ref[0] + deg_ref[1] - 1.0
    dis = lax.rsqrt(deg)
    dis_ref[...] = dis
    g_ref[...] = dis * h_ref[...]


def _tc_scale(deg2v, hv):
    return pl.pallas_call(
        _s1_body,
        grid=(GRID_TC,),
        in_specs=[pl.BlockSpec((NC, ROWS8, 128), lambda i: (0, i, 0)),
                  pl.BlockSpec((ROWS8, 128), lambda i: (i, 0))],
        out_specs=[pl.BlockSpec((ROWS8, 128), lambda i: (i, 0)),
                   pl.BlockSpec((ROWS8, 128), lambda i: (i, 0))],
        out_shape=[jax.ShapeDtypeStruct((NP8, 128), jnp.float32),
                   jax.ShapeDtypeStruct((NP8, 128), jnp.float32)],
    )(deg2v, hv)


def _s2_body(a_ref, g1_ref, dis_ref, w2_ref, b1_ref, g2_ref):
    dis = dis_ref[...]
    pre = dis * (a_ref[0] + a_ref[1] - g1_ref[...]) + b1_ref[...]
    h = jnp.maximum(pre, 0.0)
    h2 = jnp.dot(h, w2_ref[...], preferred_element_type=jnp.float32)
    g2_ref[...] = dis * h2


def _tc_layer2(agg1v, g1v, dis, W2bd, b1t):
    return pl.pallas_call(
        _s2_body,
        grid=(GRID_TC,),
        in_specs=[pl.BlockSpec((NC, ROWS8, 128), lambda i: (0, i, 0)),
                  pl.BlockSpec((ROWS8, 128), lambda i: (i, 0)),
                  pl.BlockSpec((ROWS8, 128), lambda i: (i, 0)),
                  pl.BlockSpec((128, 128), lambda i: (0, 0)),
                  pl.BlockSpec((1, 128), lambda i: (0, 0))],
        out_specs=pl.BlockSpec((ROWS8, 128), lambda i: (i, 0)),
        out_shape=jax.ShapeDtypeStruct((NP8, 128), jnp.float32),
    )(agg1v, g1v, dis, W2bd, b1t)


def _s3_body(a_ref, g2_ref, dis_ref, b2_ref, seg_ref, o_ref):
    logits = (dis_ref[...] * (a_ref[0] + a_ref[1] - g2_ref[...])
              + b2_ref[...])
    col = lax.broadcasted_iota(jnp.int32, logits.shape, 1)
    masked = jnp.where(col % H < C, logits, -jnp.inf)
    m = jnp.max(masked, axis=1, keepdims=True)
    e = jnp.exp(masked - m)
    s = jnp.dot(e, seg_ref[...], preferred_element_type=jnp.float32)
    o_ref[...] = logits - (m + jnp.log(s))


def _tc_logsoftmax(agg2v, g2v, dis, b2t, seg):
    return pl.pallas_call(
        _s3_body,
        grid=(GRID_TC,),
        in_specs=[pl.BlockSpec((NC, ROWS8, 128), lambda i: (0, i, 0)),
                  pl.BlockSpec((ROWS8, 128), lambda i: (i, 0)),
                  pl.BlockSpec((ROWS8, 128), lambda i: (i, 0)),
                  pl.BlockSpec((1, 128), lambda i: (0, 0)),
                  pl.BlockSpec((128, 128), lambda i: (0, 0))],
        out_specs=pl.BlockSpec((ROWS8, 128), lambda i: (i, 0)),
        out_shape=jax.ShapeDtypeStruct((NP8, 128), jnp.float32),
    )(agg2v, g2v, dis, b2t, seg)


def kernel(x, edge_index, W1, b1, W2, b2):
    ei32 = edge_index.astype(jnp.int32)

    x16 = jnp.pad(x.astype(jnp.bfloat16), ((0, N_PAD - N), (0, 0)))
    xs = x16.reshape(NP8, 8 * DF)
    eye8 = jnp.eye(8, dtype=jnp.float32)
    W1s = jnp.kron(eye8, W1).astype(jnp.bfloat16)
    W2p = jnp.zeros((H, H), jnp.float32).at[:, :C].set(W2)
    W2bd = jnp.kron(eye8, W2p)
    ones2 = jnp.ones((N_PAD, H), jnp.float32)
    b1t = jnp.tile(b1, 8).reshape(1, 128)
    b2t = jnp.tile(jnp.zeros((H,), jnp.float32).at[:C].set(b2), 8)
    b2t = b2t.reshape(1, 128)
    lane = jnp.arange(128)
    seg = (lane[:, None] // H == lane[None, :] // H).astype(jnp.float32)

    deg2 = _sc_deg(ei32, ones2)
    hv = _tc_mm(xs, W1s)
    deg2v = deg2.reshape(NC, NP8, 128)
    g1v, dis = _tc_scale(deg2v, hv)
    agg1 = _sc_agg(g1v.reshape(N_PAD, H), ei32)
    g2v = _tc_layer2(agg1.reshape(NC, NP8, 128), g1v, dis, W2bd, b1t)
    agg2 = _sc_agg(g2v.reshape(N_PAD, H), ei32)
    out128 = _tc_logsoftmax(agg2.reshape(NC, NP8, 128), g2v, dis, b2t, seg)
    return out128.reshape(N_PAD, H)[:N, :C]
